# Initial kernel scaffold; baseline (speedup 1.0000x reference)
import jax
import jax.numpy as jnp
from jax.experimental import pallas as pl


def kernel(q_sub, q_rel, hidden, edges, rela_embed, Ws_attn, Wr_attn, Wqr_attn, b_qr, w_alpha, b_alpha, W_h):
    raise NotImplementedError("write your pallas kernel here")


# speedup vs baseline: 2.8564x; 2.8564x over previous
import jax
import jax.numpy as jnp
from jax import lax
from jax.experimental import pallas as pl
from jax.experimental.pallas import tpu as pltpu
from jax.experimental.pallas import tpu_sc as plsc

N_NODE = 10000
N_EDGE = 320000
D = 128
ATTN = 16
B = 512
NRV = 23

NC = 2
NS = 16
NW = NC * NS
EPW = N_EDGE // NW
CHUNK = 128
NFULL = EPW // CHUNK
TAIL = EPW - NFULL * CHUNK
N_PAD = 10240
RPT = N_PAD // NS


def _pre_body(hid, rela, qrel, ws, wr, wqr, bqr, an_out, ar_out, aq_out):
    an_out[...] = jnp.dot(hid[...], ws[...].T, preferred_element_type=jnp.float32)
    ar_out[...] = jnp.dot(rela[...], wr[...].T, preferred_element_type=jnp.float32)
    qproj = jnp.dot(rela[...], wqr[...].T, preferred_element_type=jnp.float32)
    cols = lax.broadcasted_iota(jnp.int32, (B, NRV), 1)
    oh = (cols == qrel[...]).astype(jnp.float32)
    aq_out[...] = (
        jnp.dot(oh, qproj, preferred_element_type=jnp.float32) + bqr[...]
    )


def _pre_tables(hidden, rela_embed, q_rel, Ws, Wr, Wqr, b_qr):
    return pl.pallas_call(
        _pre_body,
        out_shape=(
            jax.ShapeDtypeStruct((N_NODE, ATTN), jnp.float32),
            jax.ShapeDtypeStruct((NRV, ATTN), jnp.float32),
            jax.ShapeDtypeStruct((B, ATTN), jnp.float32),
        ),
    )(hidden, rela_embed, q_rel.reshape(B, 1), Ws, Wr, Wqr,
      b_qr.reshape(1, ATTN))


def _edge_block(n, off, sub_hbm, rel_hbm, obj_hbm, ridx_hbm, hidden_hbm,
                an_hbm, acc_sh, sub_v, rel_v, obj_v, ridx_v, an_v, hs_v,
                ar_f, aq_f, rela_f, w_vec, bal, sem):
    pltpu.sync_copy(sub_hbm.at[pl.ds(off, n)], sub_v)
    pltpu.sync_copy(rel_hbm.at[pl.ds(off, n)], rel_v)
    pltpu.sync_copy(obj_hbm.at[pl.ds(off, n)], obj_v)
    pltpu.sync_copy(ridx_hbm.at[pl.ds(off, n)], ridx_v)
    cp_an = pltpu.async_copy(an_hbm.at[sub_v], an_v, sem)
    cp_hs = pltpu.async_copy(hidden_hbm.at[sub_v], hs_v, sem)
    cp_an.wait()
    cp_hs.wait()

    lane = lax.iota(jnp.int32, 16)

    def edge_body(e, _):
        e_bc = jnp.full((16,), e, jnp.int32)
        rel_bc = plsc.load_gather(rel_v, [e_bc])
        rid_bc = plsc.load_gather(ridx_v, [e_bc])
        a1 = an_v.at[e][...]
        a2 = plsc.load_gather(ar_f, [rel_bc * 16 + lane])
        a3 = plsc.load_gather(aq_f, [rid_bc * 16 + lane])
        attn = jnp.maximum(a1 + a2 + a3, 0.0)
        s = jnp.sum(attn * w_vec)
        alpha = 1.0 / (1.0 + jnp.exp(-(bal + jnp.full((16,), s))))
        hrow = hs_v.at[e]
        rel_base = rel_bc * 128 + lane
        for j in range(D // 16):
            sl = pl.ds(j * 16, 16)
            h = hrow[sl]
            r = plsc.load_gather(rela_f, [rel_base + j * 16])
            hrow[sl] = alpha * (h + r)
        return 0

    lax.fori_loop(0, n, edge_body, 0)
    pltpu.sync_copy(hs_v, acc_sh.at[obj_v], add=True)


def _sc_body(sub_hbm, rel_hbm, obj_hbm, ridx_hbm, hidden_hbm, an_hbm,
             ar_hbm, aq_hbm, rela_hbm, wal_hbm, bal_hbm, zeros_hbm,
             out_hbm,
             acc_sh, sub_v, rel_v, obj_v, ridx_v, an_v, hs_v,
             sub_t, rel_t, obj_t, ridx_t, an_t, hs_t,
             ar_f, aq_f, rela_f, wal_v, bal_v, sem):
    cid = lax.axis_index("c")
    sid = lax.axis_index("s")
    wid = cid * NS + sid

    pltpu.sync_copy(zeros_hbm.at[pl.ds(sid * RPT, RPT)],
                    acc_sh.at[pl.ds(sid * RPT, RPT)])
    pltpu.sync_copy(ar_hbm, ar_f)
    pltpu.sync_copy(aq_hbm, aq_f)
    pltpu.sync_copy(rela_hbm, rela_f)
    pltpu.sync_copy(wal_hbm, wal_v)
    pltpu.sync_copy(bal_hbm, bal_v)
    plsc.subcore_barrier()

    w_vec = wal_v[...]
    bal = bal_v[...]
    base_e = wid * EPW

    def chunk_body(i, _):
        _edge_block(CHUNK, base_e + i * CHUNK, sub_hbm, rel_hbm, obj_hbm,
                    ridx_hbm, hidden_hbm, an_hbm, acc_sh, sub_v, rel_v,
                    obj_v, ridx_v, an_v, hs_v, ar_f, aq_f, rela_f, w_vec,
                    bal, sem)
        return 0

    lax.fori_loop(0, NFULL, chunk_body, 0)
    if TAIL:
        _edge_block(TAIL, base_e + NFULL * CHUNK, sub_hbm, rel_hbm,
                    obj_hbm, ridx_hbm, hidden_hbm, an_hbm, acc_sh, sub_t,
                    rel_t, obj_t, ridx_t, an_t, hs_t, ar_f, aq_f, rela_f,
                    w_vec, bal, sem)

    plsc.subcore_barrier()
    pltpu.sync_copy(acc_sh.at[pl.ds(sid * RPT, RPT)],
                    out_hbm.at[pl.ds(cid * N_PAD + sid * RPT, RPT)])


def _sc_edges_fn():
  return pl.kernel(
    _sc_body,
    out_type=jax.ShapeDtypeStruct((NC * N_PAD, D), jnp.float32),
    compiler_params=pltpu.CompilerParams(needs_layout_passes=False,
                                         use_tc_tiling_on_sc=False),
    mesh=plsc.VectorSubcoreMesh(core_axis_name="c", subcore_axis_name="s",
                                num_cores=NC, num_subcores=NS),
    scratch_types=[
        pltpu.VMEM_SHARED((N_PAD, D), jnp.float32),
        pltpu.VMEM((CHUNK,), jnp.int32),
        pltpu.VMEM((CHUNK,), jnp.int32),
        pltpu.VMEM((CHUNK,), jnp.int32),
        pltpu.VMEM((CHUNK,), jnp.int32),
        pltpu.VMEM((CHUNK, ATTN), jnp.float32),
        pltpu.VMEM((CHUNK, D), jnp.float32),
        pltpu.VMEM((TAIL,), jnp.int32),
        pltpu.VMEM((TAIL,), jnp.int32),
        pltpu.VMEM((TAIL,), jnp.int32),
        pltpu.VMEM((TAIL,), jnp.int32),
        pltpu.VMEM((TAIL, ATTN), jnp.float32),
        pltpu.VMEM((TAIL, D), jnp.float32),
        pltpu.VMEM((NRV * ATTN,), jnp.float32),
        pltpu.VMEM((B * ATTN,), jnp.float32),
        pltpu.VMEM((NRV * D,), jnp.float32),
        pltpu.VMEM((16,), jnp.float32),
        pltpu.VMEM((16,), jnp.float32),
        pltpu.SemaphoreType.DMA,
    ],
  )


def _post_body(p0, p1, wh, out):
    out[...] = jnp.dot(p0[...] + p1[...], wh[...].T,
                       preferred_element_type=jnp.float32)


def _post(partials, W_h):
    rows = 2000
    return pl.pallas_call(
        _post_body,
        grid=(N_NODE // rows,),
        in_specs=[
            pl.BlockSpec((rows, D), lambda i: (i, 0)),
            pl.BlockSpec((rows, D), lambda i: (i, 0)),
            pl.BlockSpec((D, D), lambda i: (0, 0)),
        ],
        out_specs=pl.BlockSpec((rows, D), lambda i: (i, 0)),
        out_shape=jax.ShapeDtypeStruct((N_NODE, D), jnp.float32),
    )(partials[:N_NODE], partials[N_PAD:N_PAD + N_NODE], W_h)


def kernel(q_sub, q_rel, hidden, edges, rela_embed, Ws_attn, Wr_attn,
           Wqr_attn, b_qr, w_alpha, b_alpha, W_h):
    del q_sub
    edges = edges.astype(jnp.int32)
    sub = edges[:, 4]
    rel = edges[:, 2]
    obj = edges[:, 5]
    ridx = edges[:, 0]
    a_node, a_rel, a_q = _pre_tables(
        hidden, rela_embed, q_rel.astype(jnp.int32), Ws_attn, Wr_attn,
        Wqr_attn, b_qr)
    wal = w_alpha.reshape(ATTN)
    bal = jnp.broadcast_to(b_alpha.reshape(1), (16,))
    zeros = jnp.zeros((N_PAD, D), jnp.float32)
    partials = _sc_edges_fn()(sub, rel, obj, ridx, hidden, a_node,
                              a_rel.reshape(-1), a_q.reshape(-1),
                              rela_embed.reshape(-1), wal, bal, zeros)
    return _post(partials, W_h)
